# baseline (speedup 1.0000x reference)
import functools

import jax
import jax.numpy as jnp
from jax import lax
from jax.experimental import pallas as pl
from jax.experimental.pallas import tpu as pltpu
from jax.experimental.pallas import tpu_sc as plsc

NC = 2
NS = 16
L = 16
NW = NC * NS

CHUNK = 128


def _make_mf_kernel(batch, hidden):
    assert batch % (NW * CHUNK) == 0
    assert hidden % L == 0
    bpw = batch // NW
    nch = bpw // CHUNK
    nh = hidden // L

    mesh = plsc.VectorSubcoreMesh(core_axis_name="c", subcore_axis_name="s")

    @functools.partial(
        pl.kernel,
        mesh=mesh,
        out_type=jax.ShapeDtypeStruct((batch,), jnp.float32),
        compiler_params=pltpu.CompilerParams(
            needs_layout_passes=False, use_tc_tiling_on_sc=False),
        scratch_types=[
            pltpu.VMEM((bpw,), jnp.int32),
            pltpu.VMEM((bpw,), jnp.int32),
            pltpu.VMEM((nch, CHUNK, hidden), jnp.float32),
            pltpu.VMEM((nch, CHUNK, hidden), jnp.float32),
            pltpu.VMEM((bpw,), jnp.float32),
            pltpu.VMEM((bpw,), jnp.float32),
            pltpu.VMEM((bpw,), jnp.float32),
            pltpu.VMEM((L,), jnp.float32),
            pltpu.SemaphoreType.DMA,
        ],
    )
    def mf(user_hbm, item_hbm, uw_hbm, iw_hbm, ub_hbm, ib_hbm, bias_hbm,
           out_hbm, uidx_v, iidx_v, urows_v, irows_v, ubias_v, ibias_v,
           out_v, bias_v, sem):
        wid = lax.axis_index("s") * NC + lax.axis_index("c")
        base = wid * bpw

        pltpu.sync_copy(user_hbm.at[pl.ds(base, bpw)], uidx_v)
        pltpu.sync_copy(item_hbm.at[pl.ds(base, bpw)], iidx_v)
        pltpu.sync_copy(bias_hbm, bias_v.at[pl.ds(0, 1)])

        copies = []
        for j in range(nch):
            idx_u = uidx_v.at[pl.ds(j * CHUNK, CHUNK)]
            idx_i = iidx_v.at[pl.ds(j * CHUNK, CHUNK)]
            copies.append(pltpu.async_copy(uw_hbm.at[idx_u], urows_v.at[j], sem))
            copies.append(pltpu.async_copy(iw_hbm.at[idx_i], irows_v.at[j], sem))
            copies.append(pltpu.async_copy(
                ub_hbm.at[idx_u], ubias_v.at[pl.ds(j * CHUNK, CHUNK)], sem))
            copies.append(pltpu.async_copy(
                ib_hbm.at[idx_i], ibias_v.at[pl.ds(j * CHUNK, CHUNK)], sem))
        for c in copies:
            c.wait()

        gb = bias_v[...][0]
        lane = lax.iota(jnp.int32, L)
        for j in range(nch):
            def grp_body(g, carry, j=j):
                gbase = g * L
                vbu = ubias_v[pl.ds(j * CHUNK + gbase, L)]
                vbi = ibias_v[pl.ds(j * CHUNK + gbase, L)]
                outvec = jnp.zeros((L,), jnp.float32)
                for k in range(L):
                    row = gbase + k
                    bu = jnp.broadcast_to(vbu[k], (L,))
                    bi = jnp.broadcast_to(vbi[k], (L,))
                    acc = ((urows_v[j, row, pl.ds(0, L)] + bu)
                           * (irows_v[j, row, pl.ds(0, L)] + bi))
                    for h in range(1, nh):
                        acc = acc + ((urows_v[j, row, pl.ds(h * L, L)] + bu)
                                     * (irows_v[j, row, pl.ds(h * L, L)] + bi))
                    outvec = jnp.where(lane == k, jnp.sum(acc) + gb, outvec)
                out_v[pl.ds(j * CHUNK + gbase, L)] = outvec
                return carry

            lax.fori_loop(0, CHUNK // L, grp_body, 0)

        pltpu.sync_copy(out_v, out_hbm.at[pl.ds(base, bpw)])

    return mf


def kernel(user, item, target, user_weight, item_weight, user_bias,
           item_bias, bias):
    del target
    mf = _make_mf_kernel(user.shape[0], user_weight.shape[1])
    return mf(user, item, user_weight, item_weight,
              user_bias.reshape(-1), item_bias.reshape(-1), bias)
